# baseline (speedup 1.0000x reference)
import functools

import jax
import jax.numpy as jnp
from jax import lax
from jax.experimental import pallas as pl
from jax.experimental.pallas import tpu as pltpu
from jax.experimental.pallas import tpu_sc as plsc

N_NODES = 10000
D = 128
E = 320000
NC = 2
NS = 16
NW = NC * NS
CPT = D // NW
L = 16
KCH = 2048
NCHK = 158
E_PAD = NCHK * KCH
GRP = KCH // L
R = 10240
RPT = R // NS
ZCH = RPT // 128
CCH = 128
CNCH = 160
CE_PAD = NS * CNCH * CCH

_mesh = plsc.VectorSubcoreMesh(
    core_axis_name="c", subcore_axis_name="s", num_cores=NC, num_subcores=NS)
_sc_params = pltpu.CompilerParams(
    use_tc_tiling_on_sc=False, needs_layout_passes=False)


def _seg_body(table_t, src_h, dst_h, zeros4r, out_t,
              tbl, acc, sb0, sb1, db0, db1, gs0, gs1):
    c = lax.axis_index("c")
    s = lax.axis_index("s")
    wid = c * NS + s
    row0 = wid * CPT

    pltpu.sync_copy(table_t.at[pl.ds(row0, CPT)], tbl)
    pltpu.sync_copy(zeros4r, acc)

    sbufs = (sb0, sb1)
    dbufs = (db0, db1)
    sems = (gs0, gs1)

    def i_issue(ci, b):
        pltpu.async_copy(src_h.at[pl.ds(ci * KCH, KCH)], sbufs[b], sems[b])
        pltpu.async_copy(dst_h.at[pl.ds(ci * KCH, KCH)], dbufs[b], sems[b])

    def i_wait(b):
        pltpu.make_async_copy(src_h.at[pl.ds(0, KCH)], sbufs[b],
                              sems[b]).wait()
        pltpu.make_async_copy(dst_h.at[pl.ds(0, KCH)], dbufs[b],
                              sems[b]).wait()

    cvecs = [jnp.full((L,), col, jnp.int32) for col in range(CPT)]

    def consume(b):
        sbuf = sbufs[b]
        dbuf = dbufs[b]

        @pl.loop(0, GRP, unroll=8)
        def _(k):
            is_v = sbuf[pl.ds(k * L, L)]
            id_v = dbuf[pl.ds(k * L, L)]
            for col in range(CPT):
                v = plsc.load_gather(tbl, [cvecs[col], is_v])
                plsc.addupdate_scatter(acc, [cvecs[col], id_v], v)

    i_issue(0, 0)

    @pl.loop(0, NCHK // 2)
    def _(p):
        ci = 2 * p
        i_wait(0)
        i_issue(ci + 1, 1)
        consume(0)
        i_wait(1)

        @pl.when(ci + 2 < NCHK)
        def _():
            i_issue(ci + 2, 0)

        consume(1)

    pltpu.sync_copy(acc, out_t.at[pl.ds(row0, CPT)])


_seg_sum_t = pl.kernel(
    _seg_body,
    out_type=jax.ShapeDtypeStruct((D, R), jnp.float32),
    mesh=_mesh,
    compiler_params=_sc_params,
    scratch_types=[
        pltpu.VMEM((CPT, N_NODES), jnp.float32),
        pltpu.VMEM((CPT, R), jnp.float32),
        pltpu.VMEM((KCH,), jnp.int32),
        pltpu.VMEM((KCH,), jnp.int32),
        pltpu.VMEM((KCH,), jnp.int32),
        pltpu.VMEM((KCH,), jnp.int32),
        pltpu.SemaphoreType.DMA,
        pltpu.SemaphoreType.DMA,
    ],
)

CSEM = 4


def _counts_body(dstb, z16, ones16, cnt_out, idx_d, cbuf, ones_v, accum_c,
                 sem):
    c = lax.axis_index("c")
    s = lax.axis_index("s")
    wid = c * NS + s
    row0 = s * RPT

    pltpu.sync_copy(dstb.at[wid], idx_d)
    pltpu.sync_copy(z16, cbuf)
    for z in range(ZCH):
        pltpu.sync_copy(cbuf, accum_c.at[pl.ds(row0 + z * 128, 128)])
    pltpu.sync_copy(ones16, ones_v)
    plsc.subcore_barrier()

    def s_issue(j):
        pltpu.async_copy(ones_v, accum_c.at[idx_d.at[j]], sem, add=True)

    def s_drain(j):
        pltpu.make_async_copy(ones_v, accum_c.at[idx_d.at[j]], sem).wait()

    for j in range(CSEM):
        s_issue(j)

    @pl.loop(CSEM, CNCH)
    def _(j):
        s_drain(j - CSEM)
        s_issue(j)

    for t in range(CSEM):
        s_drain(CNCH - CSEM + t)

    plsc.subcore_barrier()
    out0 = c * R + row0
    for z in range(ZCH):
        pltpu.sync_copy(accum_c.at[pl.ds(row0 + z * 128, 128)], cbuf)
        pltpu.sync_copy(cbuf, cnt_out.at[pl.ds(out0 + z * 128, 128)])


_counts = pl.kernel(
    _counts_body,
    out_type=jax.ShapeDtypeStruct((NC * R, 16), jnp.float32),
    mesh=_mesh,
    compiler_params=_sc_params,
    scratch_types=[
        pltpu.VMEM((CNCH, CCH), jnp.int32),
        pltpu.VMEM((128, 16), jnp.float32),
        pltpu.VMEM((128, 16), jnp.float32),
        pltpu.VMEM_SHARED((R, 16), jnp.float32),
        pltpu.SemaphoreType.DMA,
    ],
)


def _dense_body(s_ref, c_ref, x_ref, wl_ref, wr_ref, b_ref, g_ref, bt_ref,
                o_ref):
    aggT = s_ref[:, 0:N_NODES] / jnp.maximum(c_ref[...], 1.0)
    xuT = (jnp.dot(wl_ref[...], aggT, preferred_element_type=jnp.float32)
           + jnp.dot(wr_ref[...], x_ref[...],
                     preferred_element_type=jnp.float32)
           + b_ref[...])
    m = jnp.mean(xuT, axis=1, keepdims=True)
    d = xuT - m
    v = jnp.mean(d * d, axis=1, keepdims=True)
    y = d * lax.rsqrt(v + 1e-5) * g_ref[...] + bt_ref[...]
    o_ref[...] = jnp.maximum(y, 0.0)


_dense_t = pl.pallas_call(
    _dense_body,
    out_shape=jax.ShapeDtypeStruct((D, N_NODES), jnp.float32),
)


def _pad_flat(idx, fill):
    pad = jnp.full((E_PAD - E,), fill, jnp.int32)
    return jnp.concatenate([idx, pad])


def _cnt_blocks(idx, fill):
    pad = jnp.full((CE_PAD - E,), fill, jnp.int32)
    return jnp.concatenate([idx, pad]).reshape(NS, CNCH, CCH)


def kernel(x_user, x_event, edge_index, params):
    u = edge_index[0].astype(jnp.int32)
    e = edge_index[1].astype(jnp.int32)
    src_u = _pad_flat(e, 0)
    dst_u = _pad_flat(u, R - 1)
    src_e = _pad_flat(u, 0)
    dst_e = _pad_flat(e, R - 1)
    dst_c = jnp.concatenate(
        [_cnt_blocks(u, R - 1), _cnt_blocks(e, R - 1)], axis=0)

    z4r = jnp.zeros((CPT, R), jnp.float32)
    z16 = jnp.zeros((128, 16), jnp.float32)
    ones16 = jnp.ones((128, 16), jnp.float32)

    cnt = _counts(dst_c, z16, ones16)
    cnt_u = cnt[0:N_NODES, 0:1].reshape(1, N_NODES)
    cnt_e = cnt[R:R + N_NODES, 0:1].reshape(1, N_NODES)

    xuT = x_user.T
    xeT = x_event.T

    def dense(S, C, xT, side, i):
        return _dense_t(S, C, xT,
                        params['Wl_%s%d' % (side, i)],
                        params['Wr_%s%d' % (side, i)],
                        params['bl_%s%d' % (side, i)].reshape(D, 1),
                        params['gamma_%s%d' % (side, i)].reshape(D, 1),
                        params['beta_%s%d' % (side, i)].reshape(D, 1))

    Su = _seg_sum_t(xeT, src_u, dst_u, z4r)
    xuT = dense(Su, cnt_u, xuT, 'u', 0)
    Se = _seg_sum_t(xuT, src_e, dst_e, z4r)
    xeT = dense(Se, cnt_e, xeT, 'e', 0)

    Su2 = _seg_sum_t(xeT, src_u, dst_u, z4r)
    xuT = dense(Su2, cnt_u, xuT, 'u', 1)
    Se2 = _seg_sum_t(xuT, src_e, dst_e, z4r)
    xeT = dense(Se2, cnt_e, xeT, 'e', 1)
    return xuT.T, xeT.T
